# baseline (speedup 1.0000x reference)
import functools

import jax
import jax.numpy as jnp
from jax import lax
from jax.experimental import pallas as pl
from jax.experimental.pallas import tpu as pltpu
from jax.experimental.pallas import tpu_sc as plsc

_INPUT_DIM = 1000000
_OUTPUT_DIM = 32
_BATCH = 16384
_N_FIELDS = 26

_NB = _BATCH * _N_FIELDS
_NC, _NS = 2, 16
_NW = _NC * _NS
_BPW = _NB // _NW
_BLK = 128
_BLKS_PER_W = _BATCH // _BLK // _NW
_UNITS = _N_FIELDS * _BLKS_PER_W
_NBUF = 8


@functools.partial(
    pl.kernel,
    out_type=jax.ShapeDtypeStruct(
        (_N_FIELDS, _OUTPUT_DIM // 8, _BATCH // _BLK, 8, _BLK), jnp.float32),
    mesh=plsc.VectorSubcoreMesh(core_axis_name="c", subcore_axis_name="s"),
    compiler_params=pltpu.CompilerParams(use_tc_tiling_on_sc=False,
                                         needs_layout_passes=False,
                                         disable_bounds_checks=True),
    scratch_types=(
        [pltpu.VMEM((_BPW,), jnp.int32)]
        + [pltpu.VMEM((_BLK,), jnp.int32) for _ in range(_NBUF)]
        + [pltpu.VMEM((_BLK, _OUTPUT_DIM), jnp.float32) for _ in range(_NBUF)]
        + [pltpu.VMEM((_OUTPUT_DIM, _BLK), jnp.float32)
           for _ in range(_NBUF)]
        + [pltpu.SemaphoreType.DMA for _ in range(2 * _NBUF)]
    ),
)
def _emb_lookup(table_hbm, idx_hbm, out_hbm, idx_v, *bufs):
    wid = lax.axis_index("s") * _NC + lax.axis_index("c")
    pltpu.sync_copy(idx_hbm.at[pl.ds(wid * _BPW, _BPW)], idx_v)

    icol = bufs[:_NBUF]
    rows = bufs[_NBUF:2 * _NBUF]
    trans = bufs[2 * _NBUF:3 * _NBUF]
    gsem = bufs[3 * _NBUF:4 * _NBUF]
    wsem = bufs[4 * _NBUF:5 * _NBUF]

    def unit_ids(u):
        if isinstance(u, int):
            return u >> 2, u & 3
        return (lax.shift_right_logical(u, 2), lax.bitwise_and(u, 3))

    def build_icol(u, p):
        b2, blk = unit_ids(u)
        base = blk * (_BLK * _N_FIELDS) + b2
        step = lax.iota(jnp.int32, 16) * _N_FIELDS
        for g in range(8):
            vals = plsc.load_gather(idx_v, [step + (base + g * 16 * _N_FIELDS)])
            icol[p][pl.ds(g * 16, 16)] = vals

    def fire_gather(p):
        pltpu.make_async_copy(table_hbm.at[icol[p]], rows[p],
                              gsem[p]).start()

    def drain_gather(p):
        pltpu.make_async_copy(table_hbm.at[pl.ds(0, _BLK)], rows[p],
                              gsem[p]).wait()

    iota = lax.iota(jnp.int32, 16)
    d_vecs = [(iota + d0) & (_OUTPUT_DIM - 1) for d0 in range(_OUTPUT_DIM)]

    def transpose(p):
        def tbody(g, carry):
            row_ids = iota + g * 16
            for d0 in range(0, _OUTPUT_DIM, 8):
                vals = [plsc.load_gather(rows[p], [row_ids, d_vecs[d0 + i]])
                        for i in range(8)]
                for i in range(8):
                    plsc.store_scatter(trans[p], [d_vecs[d0 + i], row_ids],
                                       vals[i])
            return carry

        lax.fori_loop(0, 8, tbody, 0)

    def fire_write(u, p):
        b2, blk = unit_ids(u)
        blkg = wid * _BLKS_PER_W + blk
        for r in range(_OUTPUT_DIM // 8):
            pltpu.make_async_copy(trans[p].at[pl.ds(r * 8, 8)],
                                  out_hbm.at[b2, r, blkg], wsem[p]).start()

    def drain_write(p):
        for r in range(_OUTPUT_DIM // 8):
            pltpu.make_async_copy(trans[p].at[pl.ds(r * 8, 8)],
                                  out_hbm.at[0, 0, 0], wsem[p]).wait()

    for p in range(_NBUF):
        build_icol(p, p)
        fire_gather(p)
    for p in range(_NBUF):
        drain_gather(p)
        transpose(p)
        fire_write(p, p)
        build_icol(p + _NBUF, p)
        fire_gather(p)

    def body(i, carry):
        for p in range(_NBUF):
            u = _NBUF * i + p
            drain_gather(p)
            drain_write(p)
            transpose(p)
            fire_write(u, p)
            build_icol(u + _NBUF, p)
            fire_gather(p)
        return carry

    lax.fori_loop(1, (_UNITS - _NBUF) // _NBUF, body, 0)

    for p in range(_NBUF):
        drain_gather(p)
        drain_write(p)
        transpose(p)
        fire_write(_UNITS - _NBUF + p, p)
    for p in range(_NBUF):
        drain_write(p)


def kernel(inputs, embeddings):
    idx = inputs.reshape(-1).astype(jnp.int32)
    out5 = _emb_lookup(embeddings, idx)
    out = out5.transpose(2, 4, 0, 1, 3).reshape(_BATCH, _N_FIELDS, _OUTPUT_DIM)
    return out


# speedup vs baseline: 1.0190x; 1.0190x over previous
import functools

import jax
import jax.numpy as jnp
from jax import lax
from jax.experimental import pallas as pl
from jax.experimental.pallas import tpu as pltpu
from jax.experimental.pallas import tpu_sc as plsc

_INPUT_DIM = 1000000
_OUTPUT_DIM = 32
_BATCH = 16384
_N_FIELDS = 26

_NB = _BATCH * _N_FIELDS
_NC, _NS = 2, 16
_NW = _NC * _NS
_BPW = _NB // _NW
_BLK = 128
_BLKS_PER_W = _BATCH // _BLK // _NW
_UNITS = _N_FIELDS * _BLKS_PER_W
_NBUF = 4


@functools.partial(
    pl.kernel,
    out_type=jax.ShapeDtypeStruct(
        (_N_FIELDS, _OUTPUT_DIM // 8, _BATCH // _BLK, 8, _BLK), jnp.float32),
    mesh=plsc.VectorSubcoreMesh(core_axis_name="c", subcore_axis_name="s"),
    compiler_params=pltpu.CompilerParams(use_tc_tiling_on_sc=False,
                                         needs_layout_passes=False,
                                         disable_bounds_checks=True),
    scratch_types=(
        [pltpu.VMEM((_BPW,), jnp.int32)]
        + [pltpu.VMEM((_BLK,), jnp.int32) for _ in range(_NBUF)]
        + [pltpu.VMEM((_BLK, _OUTPUT_DIM), jnp.float32) for _ in range(_NBUF)]
        + [pltpu.VMEM((_OUTPUT_DIM, _BLK), jnp.float32)
           for _ in range(_NBUF)]
        + [pltpu.SemaphoreType.DMA for _ in range(2 * _NBUF)]
    ),
)
def _emb_lookup(table_hbm, idx_hbm, out_hbm, idx_v, *bufs):
    wid = lax.axis_index("s") * _NC + lax.axis_index("c")
    pltpu.sync_copy(idx_hbm.at[pl.ds(wid * _BPW, _BPW)], idx_v)

    icol = bufs[:_NBUF]
    rows = bufs[_NBUF:2 * _NBUF]
    trans = bufs[2 * _NBUF:3 * _NBUF]
    gsem = bufs[3 * _NBUF:4 * _NBUF]
    wsem = bufs[4 * _NBUF:5 * _NBUF]

    def unit_ids(u):
        if isinstance(u, int):
            return u >> 2, u & 3
        return (lax.shift_right_logical(u, 2), lax.bitwise_and(u, 3))

    def build_icol(u, p):
        b2, blk = unit_ids(u)
        base = blk * (_BLK * _N_FIELDS) + b2
        step = lax.iota(jnp.int32, 16) * _N_FIELDS
        for g in range(8):
            vals = plsc.load_gather(idx_v, [step + (base + g * 16 * _N_FIELDS)])
            icol[p][pl.ds(g * 16, 16)] = vals

    def fire_gather(p):
        pltpu.make_async_copy(table_hbm.at[icol[p]], rows[p],
                              gsem[p]).start()

    def drain_gather(p):
        pltpu.make_async_copy(table_hbm.at[pl.ds(0, _BLK)], rows[p],
                              gsem[p]).wait()

    iota = lax.iota(jnp.int32, 16)
    d_vecs = [(iota + d0) & (_OUTPUT_DIM - 1) for d0 in range(_OUTPUT_DIM)]

    def transpose(p):
        def tbody(g, carry):
            row_ids = iota + g * 16
            for d0 in range(0, _OUTPUT_DIM, 8):
                vals = [plsc.load_gather(rows[p], [row_ids, d_vecs[d0 + i]])
                        for i in range(8)]
                for i in range(8):
                    plsc.store_scatter(trans[p], [d_vecs[d0 + i], row_ids],
                                       vals[i])
            return carry

        lax.fori_loop(0, 8, tbody, 0)

    def fire_write(u, p):
        b2, blk = unit_ids(u)
        blkg = wid * _BLKS_PER_W + blk
        for r in range(_OUTPUT_DIM // 8):
            pltpu.make_async_copy(trans[p].at[pl.ds(r * 8, 8)],
                                  out_hbm.at[b2, r, blkg], wsem[p]).start()

    def drain_write(p):
        for r in range(_OUTPUT_DIM // 8):
            pltpu.make_async_copy(trans[p].at[pl.ds(r * 8, 8)],
                                  out_hbm.at[0, 0, 0], wsem[p]).wait()

    for p in range(_NBUF):
        build_icol(p, p)
        fire_gather(p)
    for p in range(_NBUF):
        drain_gather(p)
        transpose(p)
        fire_write(p, p)
        build_icol(p + _NBUF, p)
        fire_gather(p)

    def body(i, carry):
        for p in range(_NBUF):
            u = _NBUF * i + p
            drain_gather(p)
            drain_write(p)
            transpose(p)
            fire_write(u, p)
            build_icol(u + _NBUF, p)
            fire_gather(p)
        return carry

    lax.fori_loop(1, (_UNITS - _NBUF) // _NBUF, body, 0)

    for p in range(_NBUF):
        drain_gather(p)
        drain_write(p)
        transpose(p)
        fire_write(_UNITS - _NBUF + p, p)
    for p in range(_NBUF):
        drain_write(p)


def kernel(inputs, embeddings):
    idx = inputs.reshape(-1).astype(jnp.int32)
    out5 = _emb_lookup(embeddings, idx)
    out = out5.transpose(2, 4, 0, 1, 3).reshape(_BATCH, _N_FIELDS, _OUTPUT_DIM)
    return out
